# baseline (speedup 1.0000x reference)
import functools
import jax
import jax.numpy as jnp
from jax import lax
from jax.experimental import pallas as pl
from jax.experimental.pallas import tpu as pltpu
from jax.experimental.pallas import tpu_sc as plsc

K_TOP_ = 32
L_ = 16
N_ = 32768
ROWS_ = 128
N_CORES_ = 2
N_SUBCORES_ = 16
N_WORKERS_ = N_CORES_ * N_SUBCORES_
RPW_ = ROWS_ // N_WORKERS_
NG_ = 32
NV_ = N_ // L_
NCELLS_ = NG_ * L_
CELL_ = N_ // NCELLS_

_GATHER_DNUMS_ = lax.GatherDimensionNumbers(
    offset_dims=(), collapsed_slice_dims=(0,), start_index_map=(0,))


def _lane_perm(v, idx):
    return lax.gather(v, idx[:, None], _GATHER_DNUMS_, slice_sizes=(1,),
                      mode=lax.GatherScatterMode.PROMISE_IN_BOUNDS,
                      unique_indices=True)


def _vmax_splat(v, iota):
    for s in (1, 2, 4, 8):
        v = jnp.maximum(v, _lane_perm(v, iota ^ s))
    return v


def _process_row(rowbuf, candbuf, blkidx, outbuf, iota, neg):
    rowbuf[pl.ds(N_, L_)] = neg

    def p1(i, accs):
        base = i * (NG_ * L_)
        return tuple(
            jnp.maximum(accs[g], rowbuf[pl.ds(base + g * L_, L_)])
            for g in range(NG_))

    accs = lax.fori_loop(0, NV_ // NG_, p1, (neg,) * NG_)

    ahalf = accs[0]
    for g in range(1, NG_ // 2):
        ahalf = jnp.maximum(ahalf, accs[g])
    bhalf = accs[NG_ // 2]
    for g in range(NG_ // 2 + 1, NG_):
        bhalf = jnp.maximum(bhalf, accs[g])
    t0v = -_vmax_splat(-jnp.minimum(ahalf, bhalf), iota)

    cnts = [jnp.sum((accs[g] >= t0v).astype(jnp.int32)) for g in range(NG_)]
    nb = jnp.int32(0)
    for g in range(NG_):
        msk = accs[g] >= t0v
        plsc.store_compressed(blkidx.at[pl.ds(nb, L_)], g * L_ + iota,
                              mask=msk)
        nb = nb + cnts[g]
    blkidx[pl.ds(nb, L_)] = jnp.full((L_,), N_, jnp.int32)

    strides = [(iota + q * L_) * NCELLS_ for q in range(CELL_ // L_)]
    nfull = (nb + (L_ - 1)) // L_

    def p2b(gi, ptr):
        idvec = blkidx[pl.ds(gi * L_, L_)]
        for j in range(L_):
            bid = idvec[j]
            vs = [plsc.load_gather(rowbuf, [jnp.minimum(bid + st, N_)])
                  for st in strides]
            ms = [v >= t0v for v in vs]
            cs = [jnp.sum(m.astype(jnp.int32)) for m in ms]
            for q in range(CELL_ // L_):
                plsc.store_compressed(candbuf.at[pl.ds(ptr, L_)], vs[q],
                                      mask=ms[q])
                ptr = ptr + cs[q]
        return ptr

    ptr = lax.fori_loop(0, nfull, p2b, jnp.int32(0))
    candbuf[pl.ds(ptr, L_)] = neg
    ncv = ptr // L_ + 1

    def pa(j, acc):
        return jnp.maximum(acc, candbuf[pl.ds(j * L_, L_)])

    def emit_cond(i):
        return i < K_TOP_

    def emit(i):
        acc = lax.fori_loop(0, ncv, pa, neg)
        mv = _vmax_splat(acc, iota)

        def pb(j, cnt):
            v = candbuf[pl.ds(j * L_, L_)]
            eq = v == mv
            candbuf[pl.ds(j * L_, L_)] = jnp.where(eq, neg, v)
            return cnt + jnp.sum(eq.astype(jnp.int32))

        cnt = lax.fori_loop(0, ncv, pb, jnp.int32(0))
        pos0 = iota + i
        plsc.store_scatter(outbuf, [pos0], mv,
                           mask=(iota < cnt) & (pos0 < K_TOP_))
        pos1 = pos0 + L_
        plsc.store_scatter(outbuf, [pos1], mv,
                           mask=((iota + L_) < cnt) & (pos1 < K_TOP_))
        return i + cnt

    lax.while_loop(emit_cond, emit, jnp.int32(0))


def _sc_topk_body(x_hbm, out_hbm, rowbuf0, rowbuf1, candbuf, blkidx, outbuf,
                  sem0, sem1):
    wid = lax.axis_index("s") * N_CORES_ + lax.axis_index("c")
    iota = lax.iota(jnp.int32, L_)
    neg = jnp.full((L_,), -jnp.inf, jnp.float32)
    row0 = wid * RPW_

    bufs = [rowbuf0, rowbuf1]
    sems = [sem0, sem1]
    cps = [None] * RPW_
    cps[0] = pltpu.async_copy(x_hbm.at[row0], rowbuf0.at[pl.ds(0, N_)], sem0)
    for r in range(RPW_):
        cps[r].wait()
        if r + 1 < RPW_:
            cps[r + 1] = pltpu.async_copy(
                x_hbm.at[row0 + r + 1],
                bufs[(r + 1) % 2].at[pl.ds(0, N_)], sems[(r + 1) % 2])
        _process_row(bufs[r % 2], candbuf, blkidx, outbuf, iota, neg)
        pltpu.sync_copy(outbuf, out_hbm.at[row0 + r])


@functools.lru_cache(maxsize=1)
def _build_sc_topk():
    return pl.kernel(
        _sc_topk_body,
        out_type=jax.ShapeDtypeStruct((ROWS_, K_TOP_), jnp.float32),
        mesh=plsc.VectorSubcoreMesh(core_axis_name="c", subcore_axis_name="s",
                                    num_cores=N_CORES_,
                                    num_subcores=N_SUBCORES_),
        scratch_types=[
            pltpu.VMEM((N_ + L_,), jnp.float32),
            pltpu.VMEM((N_ + L_,), jnp.float32),
            pltpu.VMEM((N_ + L_,), jnp.float32),
            pltpu.VMEM((NCELLS_ + L_,), jnp.int32),
            pltpu.VMEM((K_TOP_,), jnp.float32),
            pltpu.SemaphoreType.DMA,
            pltpu.SemaphoreType.DMA,
        ],
        compiler_params=pltpu.CompilerParams(needs_layout_passes=False),
    )


def kernel(inputs):
    return _build_sc_topk()(inputs)


# speedup vs baseline: 1.0009x; 1.0009x over previous
import functools
import jax
import jax.numpy as jnp
from jax import lax
from jax.experimental import pallas as pl
from jax.experimental.pallas import tpu as pltpu
from jax.experimental.pallas import tpu_sc as plsc

K_TOP_ = 32
L_ = 16
N_ = 32768
ROWS_ = 128
N_CORES_ = 2
N_SUBCORES_ = 16
N_WORKERS_ = N_CORES_ * N_SUBCORES_
RPW_ = ROWS_ // N_WORKERS_
NG_ = 32
NV_ = N_ // L_
NCELLS_ = NG_ * L_
CELL_ = N_ // NCELLS_

_GATHER_DNUMS_ = lax.GatherDimensionNumbers(
    offset_dims=(), collapsed_slice_dims=(0,), start_index_map=(0,))


def _lane_perm(v, idx):
    return lax.gather(v, idx[:, None], _GATHER_DNUMS_, slice_sizes=(1,),
                      mode=lax.GatherScatterMode.PROMISE_IN_BOUNDS,
                      unique_indices=True)


def _vmax_splat(v, iota):
    for s in (1, 2, 4, 8):
        v = jnp.maximum(v, _lane_perm(v, iota ^ s))
    return v


def _process_row(rowbuf, candbuf, blkidx, outbuf, iota, neg):
    rowbuf[pl.ds(N_, L_)] = neg

    half_words = (NV_ // 2) * L_

    def p1_pass(h):
        def p1(i, accs):
            base = h * half_words + i * (L_ * L_)
            return tuple(
                jnp.maximum(accs[g], rowbuf[pl.ds(base + g * L_, L_)])
                for g in range(L_))

        return lax.fori_loop(0, NV_ // 2 // L_, p1, (neg,) * L_)

    accs_a = p1_pass(0)
    accs_b = p1_pass(1)
    accs = list(accs_a) + list(accs_b)

    ahalf = accs_a[0]
    for g in range(1, L_):
        ahalf = jnp.maximum(ahalf, accs_a[g])
    bhalf = accs_b[0]
    for g in range(1, L_):
        bhalf = jnp.maximum(bhalf, accs_b[g])
    t0v = -_vmax_splat(-jnp.minimum(ahalf, bhalf), iota)

    cnts = [jnp.sum((accs[g] >= t0v).astype(jnp.int32)) for g in range(NG_)]
    nb = jnp.int32(0)
    for g in range(NG_):
        msk = accs[g] >= t0v
        plsc.store_compressed(blkidx.at[pl.ds(nb, L_)], g * L_ + iota,
                              mask=msk)
        nb = nb + cnts[g]
    blkidx[pl.ds(nb, L_)] = jnp.full((L_,), N_, jnp.int32)

    stride_w = NCELLS_ // 2
    strides = [(iota + q * L_) * stride_w for q in range(CELL_ // L_)]
    nfull = (nb + (L_ - 1)) // L_

    def p2b(gi, ptr):
        idvec = blkidx[pl.ds(gi * L_, L_)]
        for j in range(L_):
            bid = idvec[j]
            base = bid + jnp.where(bid >= stride_w,
                                   half_words - stride_w, 0)
            vs = [plsc.load_gather(rowbuf, [jnp.minimum(base + st, N_)])
                  for st in strides]
            ms = [v >= t0v for v in vs]
            cs = [jnp.sum(m.astype(jnp.int32)) for m in ms]
            for q in range(CELL_ // L_):
                plsc.store_compressed(candbuf.at[pl.ds(ptr, L_)], vs[q],
                                      mask=ms[q])
                ptr = ptr + cs[q]
        return ptr

    ptr = lax.fori_loop(0, nfull, p2b, jnp.int32(0))
    candbuf[pl.ds(ptr, L_)] = neg
    ncv = ptr // L_ + 1

    def pa(j, acc):
        return jnp.maximum(acc, candbuf[pl.ds(j * L_, L_)])

    def emit_cond(i):
        return i < K_TOP_

    def emit(i):
        acc = lax.fori_loop(0, ncv, pa, neg)
        mv = _vmax_splat(acc, iota)

        def pb(j, cnt):
            v = candbuf[pl.ds(j * L_, L_)]
            eq = v == mv
            candbuf[pl.ds(j * L_, L_)] = jnp.where(eq, neg, v)
            return cnt + jnp.sum(eq.astype(jnp.int32))

        cnt = lax.fori_loop(0, ncv, pb, jnp.int32(0))
        pos0 = iota + i
        plsc.store_scatter(outbuf, [pos0], mv,
                           mask=(iota < cnt) & (pos0 < K_TOP_))
        pos1 = pos0 + L_
        plsc.store_scatter(outbuf, [pos1], mv,
                           mask=((iota + L_) < cnt) & (pos1 < K_TOP_))
        return i + cnt

    lax.while_loop(emit_cond, emit, jnp.int32(0))


def _sc_topk_body(x_hbm, out_hbm, rowbuf0, rowbuf1, candbuf, blkidx, outbuf,
                  sem0, sem1):
    wid = lax.axis_index("s") * N_CORES_ + lax.axis_index("c")
    iota = lax.iota(jnp.int32, L_)
    neg = jnp.full((L_,), -jnp.inf, jnp.float32)
    row0 = wid * RPW_

    bufs = [rowbuf0, rowbuf1]
    sems = [sem0, sem1]
    cps = [None] * RPW_
    cps[0] = pltpu.async_copy(x_hbm.at[row0], rowbuf0.at[pl.ds(0, N_)], sem0)
    for r in range(RPW_):
        cps[r].wait()
        if r + 1 < RPW_:
            cps[r + 1] = pltpu.async_copy(
                x_hbm.at[row0 + r + 1],
                bufs[(r + 1) % 2].at[pl.ds(0, N_)], sems[(r + 1) % 2])
        _process_row(bufs[r % 2], candbuf, blkidx, outbuf, iota, neg)
        pltpu.sync_copy(outbuf, out_hbm.at[row0 + r])


@functools.lru_cache(maxsize=1)
def _build_sc_topk():
    return pl.kernel(
        _sc_topk_body,
        out_type=jax.ShapeDtypeStruct((ROWS_, K_TOP_), jnp.float32),
        mesh=plsc.VectorSubcoreMesh(core_axis_name="c", subcore_axis_name="s",
                                    num_cores=N_CORES_,
                                    num_subcores=N_SUBCORES_),
        scratch_types=[
            pltpu.VMEM((N_ + L_,), jnp.float32),
            pltpu.VMEM((N_ + L_,), jnp.float32),
            pltpu.VMEM((N_ + L_,), jnp.float32),
            pltpu.VMEM((NCELLS_ + L_,), jnp.int32),
            pltpu.VMEM((K_TOP_,), jnp.float32),
            pltpu.SemaphoreType.DMA,
            pltpu.SemaphoreType.DMA,
        ],
        compiler_params=pltpu.CompilerParams(needs_layout_passes=False),
    )


def kernel(inputs):
    return _build_sc_topk()(inputs)


# speedup vs baseline: 1.6522x; 1.6507x over previous
import functools
import jax
import jax.numpy as jnp
from jax import lax
from jax.experimental import pallas as pl
from jax.experimental.pallas import tpu as pltpu
from jax.experimental.pallas import tpu_sc as plsc

K_TOP_ = 32
L_ = 16
N_ = 32768
ROWS_ = 128
N_CORES_ = 2
N_SUBCORES_ = 16
N_WORKERS_ = N_CORES_ * N_SUBCORES_
RPW_ = ROWS_ // N_WORKERS_
NG_ = 32
NV_ = N_ // L_
NCELLS_ = NG_ * L_
CELL_ = N_ // NCELLS_

_GATHER_DNUMS_ = lax.GatherDimensionNumbers(
    offset_dims=(), collapsed_slice_dims=(0,), start_index_map=(0,))


def _lane_perm(v, idx):
    return lax.gather(v, idx[:, None], _GATHER_DNUMS_, slice_sizes=(1,),
                      mode=lax.GatherScatterMode.PROMISE_IN_BOUNDS,
                      unique_indices=True)


def _vmax_splat(v, iota):
    for s in (1, 2, 4, 8):
        v = jnp.maximum(v, _lane_perm(v, iota ^ s))
    return v


def _process_row(rowbuf, candbuf, blkidx, outbuf, iota, neg):
    rowbuf[pl.ds(N_, L_)] = neg

    half_words = (NV_ // 2) * L_

    def p1_pass(h):
        def p1(i, accs):
            base = h * half_words + i * (L_ * L_)
            return tuple(
                jnp.maximum(accs[g], rowbuf[pl.ds(base + g * L_, L_)])
                for g in range(L_))

        return lax.fori_loop(0, NV_ // 2 // L_, p1, (neg,) * L_)

    accs_a = p1_pass(0)
    accs_b = p1_pass(1)
    accs = list(accs_a) + list(accs_b)

    ahalf = accs_a[0]
    for g in range(1, L_):
        ahalf = jnp.maximum(ahalf, accs_a[g])
    bhalf = accs_b[0]
    for g in range(1, L_):
        bhalf = jnp.maximum(bhalf, accs_b[g])
    t0v = -_vmax_splat(-jnp.minimum(ahalf, bhalf), iota)

    cnts = [jnp.sum((accs[g] >= t0v).astype(jnp.int32)) for g in range(NG_)]
    nb = jnp.int32(0)
    for g in range(NG_):
        msk = accs[g] >= t0v
        plsc.store_compressed(blkidx.at[pl.ds(nb, L_)], g * L_ + iota,
                              mask=msk)
        nb = nb + cnts[g]
    blkidx[pl.ds(nb, L_)] = jnp.full((L_,), N_, jnp.int32)

    stride_w = NCELLS_ // 2
    U2B = 4
    nfull = (nb + (L_ - 1)) // L_

    def p2b(gi, ptr):
        idvec = blkidx[pl.ds(gi * L_, L_)]
        base0 = idvec + jnp.where(idvec >= stride_w,
                                  half_words - stride_w, 0)

        def inner(k4, carry):
            ptr, base = carry
            idxs = [jnp.minimum(base + q * stride_w, N_) for q in range(U2B)]
            vs = [plsc.load_gather(rowbuf, [ix]) for ix in idxs]
            ms = [v >= t0v for v in vs]
            cs = [jnp.sum(m.astype(jnp.int32)) for m in ms]
            for q in range(U2B):
                plsc.store_compressed(candbuf.at[pl.ds(ptr, L_)], vs[q],
                                      mask=ms[q])
                ptr = ptr + cs[q]
            return ptr, base + U2B * stride_w

        ptr, _ = lax.fori_loop(0, CELL_ // U2B, inner, (ptr, base0))
        return ptr

    ptr = lax.fori_loop(0, nfull, p2b, jnp.int32(0))
    candbuf[pl.ds(ptr, L_)] = neg
    ncv = ptr // L_ + 1

    def pa(j, acc):
        return jnp.maximum(acc, candbuf[pl.ds(j * L_, L_)])

    mv0 = _vmax_splat(lax.fori_loop(0, ncv, pa, neg), iota)

    def emit_cond(carry):
        return carry[0] < K_TOP_

    def emit(carry):
        i, mv = carry

        def pb(j, c):
            cntv, nxt = c
            v = candbuf[pl.ds(j * L_, L_)]
            eq = v == mv
            vnew = jnp.where(eq, neg, v)
            candbuf[pl.ds(j * L_, L_)] = vnew
            return cntv + eq.astype(jnp.int32), jnp.maximum(nxt, vnew)

        cntv, nxt = lax.fori_loop(0, ncv, pb,
                                  (jnp.zeros((L_,), jnp.int32), neg))
        cnt = jnp.sum(cntv)
        pos0 = iota + i
        plsc.store_scatter(outbuf, [pos0], mv,
                           mask=(iota < cnt) & (pos0 < K_TOP_))
        pos1 = pos0 + L_
        plsc.store_scatter(outbuf, [pos1], mv,
                           mask=((iota + L_) < cnt) & (pos1 < K_TOP_))
        return i + cnt, _vmax_splat(nxt, iota)

    lax.while_loop(emit_cond, emit, (jnp.int32(0), mv0))


def _sc_topk_body(x_hbm, out_hbm, rowbuf0, rowbuf1, candbuf, blkidx, outbuf,
                  sem0, sem1):
    wid = lax.axis_index("s") * N_CORES_ + lax.axis_index("c")
    iota = lax.iota(jnp.int32, L_)
    neg = jnp.full((L_,), -jnp.inf, jnp.float32)
    row0 = wid * RPW_

    bufs = [rowbuf0, rowbuf1]
    sems = [sem0, sem1]
    cps = [None] * RPW_
    cps[0] = pltpu.async_copy(x_hbm.at[row0], rowbuf0.at[pl.ds(0, N_)], sem0)
    for r in range(RPW_):
        cps[r].wait()
        if r + 1 < RPW_:
            cps[r + 1] = pltpu.async_copy(
                x_hbm.at[row0 + r + 1],
                bufs[(r + 1) % 2].at[pl.ds(0, N_)], sems[(r + 1) % 2])
        _process_row(bufs[r % 2], candbuf, blkidx, outbuf, iota, neg)
        pltpu.sync_copy(outbuf, out_hbm.at[row0 + r])


@functools.lru_cache(maxsize=1)
def _build_sc_topk():
    return pl.kernel(
        _sc_topk_body,
        out_type=jax.ShapeDtypeStruct((ROWS_, K_TOP_), jnp.float32),
        mesh=plsc.VectorSubcoreMesh(core_axis_name="c", subcore_axis_name="s",
                                    num_cores=N_CORES_,
                                    num_subcores=N_SUBCORES_),
        scratch_types=[
            pltpu.VMEM((N_ + L_,), jnp.float32),
            pltpu.VMEM((N_ + L_,), jnp.float32),
            pltpu.VMEM((N_ + L_,), jnp.float32),
            pltpu.VMEM((NCELLS_ + L_,), jnp.int32),
            pltpu.VMEM((K_TOP_,), jnp.float32),
            pltpu.SemaphoreType.DMA,
            pltpu.SemaphoreType.DMA,
        ],
        compiler_params=pltpu.CompilerParams(needs_layout_passes=False),
    )


def kernel(inputs):
    return _build_sc_topk()(inputs)


# speedup vs baseline: 2.0909x; 1.2656x over previous
import functools
import jax
import jax.numpy as jnp
from jax import lax
from jax.experimental import pallas as pl
from jax.experimental.pallas import tpu as pltpu
from jax.experimental.pallas import tpu_sc as plsc

K_TOP_ = 32
L_ = 16
N_ = 32768
ROWS_ = 128
N_CORES_ = 2
N_SUBCORES_ = 16
N_WORKERS_ = N_CORES_ * N_SUBCORES_
RPW_ = ROWS_ // N_WORKERS_
NG_ = 32
NV_ = N_ // L_
NCELLS_ = NG_ * L_
CELL_ = N_ // NCELLS_

_GATHER_DNUMS_ = lax.GatherDimensionNumbers(
    offset_dims=(), collapsed_slice_dims=(0,), start_index_map=(0,))


def _lane_perm(v, idx):
    return lax.gather(v, idx[:, None], _GATHER_DNUMS_, slice_sizes=(1,),
                      mode=lax.GatherScatterMode.PROMISE_IN_BOUNDS,
                      unique_indices=True)


def _vmax_splat(v, iota):
    for s in (1, 2, 4, 8):
        v = jnp.maximum(v, _lane_perm(v, iota ^ s))
    return v


def _process_row(rowbuf, candbuf, blkidx, outbuf, rslot, iota, neg):
    rowbuf[pl.ds(N_, L_)] = neg

    half_words = (NV_ // 2) * L_

    def p1_pass(h):
        def p1(i, accs):
            base = h * half_words + i * (L_ * L_)
            return tuple(
                jnp.maximum(accs[g], rowbuf[pl.ds(base + g * L_, L_)])
                for g in range(L_))

        return lax.fori_loop(0, NV_ // 2 // L_, p1, (neg,) * L_)

    accs_a = p1_pass(0)
    accs_b = p1_pass(1)
    accs = list(accs_a) + list(accs_b)

    ahalf = accs_a[0]
    for g in range(1, L_):
        ahalf = jnp.maximum(ahalf, accs_a[g])
    bhalf = accs_b[0]
    for g in range(1, L_):
        bhalf = jnp.maximum(bhalf, accs_b[g])
    t0v = -_vmax_splat(-jnp.minimum(ahalf, bhalf), iota)

    thi = _vmax_splat(jnp.maximum(ahalf, bhalf), iota)
    k32 = jnp.int32(K_TOP_)
    for _ in range(8):
        tmid = 0.5 * (t0v + thi)
        cacc = jnp.zeros((L_,), jnp.int32)
        for g in range(NG_):
            cacc = cacc + (accs[g] >= tmid).astype(jnp.int32)
        ok = jnp.sum(cacc) >= k32
        t0v = jnp.where(ok, tmid, t0v)
        thi = jnp.where(ok, thi, tmid)

    cnts = [jnp.sum((accs[g] >= t0v).astype(jnp.int32)) for g in range(NG_)]
    nb = jnp.int32(0)
    for g in range(NG_):
        msk = accs[g] >= t0v
        plsc.store_compressed(blkidx.at[pl.ds(nb, L_)], g * L_ + iota,
                              mask=msk)
        nb = nb + cnts[g]
    blkidx[pl.ds(nb, L_)] = jnp.full((L_,), N_, jnp.int32)

    stride_w = NCELLS_ // 2
    U2B = 4
    nfull = (nb + (L_ - 1)) // L_

    def p2b(gi, ptr):
        idvec = blkidx[pl.ds(gi * L_, L_)]
        base0 = idvec + jnp.where(idvec >= stride_w,
                                  half_words - stride_w, 0)

        def inner(k4, carry):
            ptr, base = carry
            idxs = [jnp.minimum(base + q * stride_w, N_) for q in range(U2B)]
            vs = [plsc.load_gather(rowbuf, [ix]) for ix in idxs]
            ms = [v >= t0v for v in vs]
            cs = [jnp.sum(m.astype(jnp.int32)) for m in ms]
            for q in range(U2B):
                plsc.store_compressed(candbuf.at[pl.ds(ptr, L_)], vs[q],
                                      mask=ms[q])
                ptr = ptr + cs[q]
            return ptr, base + U2B * stride_w

        ptr, _ = lax.fori_loop(0, CELL_ // U2B, inner, (ptr, base0))
        return ptr

    ptr = lax.fori_loop(0, nfull, p2b, jnp.int32(0))
    candbuf[pl.ds(ptr, L_)] = neg
    ncv = ptr // L_ + 1

    def pa(j, acc):
        return jnp.maximum(acc, candbuf[pl.ds(j * L_, L_)])

    mv0 = _vmax_splat(lax.fori_loop(0, ncv, pa, neg), iota)

    def emit_cond(carry):
        return carry[0] < K_TOP_

    def emit(carry):
        i, mv = carry

        def pb(j, c):
            cntv, nxt = c
            v = candbuf[pl.ds(j * L_, L_)]
            eq = v == mv
            vnew = jnp.where(eq, neg, v)
            candbuf[pl.ds(j * L_, L_)] = vnew
            return cntv + eq.astype(jnp.int32), jnp.maximum(nxt, vnew)

        cntv, nxt = lax.fori_loop(0, ncv, pb,
                                  (jnp.zeros((L_,), jnp.int32), neg))
        cnt = jnp.sum(cntv)
        rvec = jnp.full((L_,), rslot, jnp.int32)
        pos0 = iota + i
        plsc.store_scatter(outbuf, [rvec, pos0], mv,
                           mask=(iota < cnt) & (pos0 < K_TOP_))
        pos1 = pos0 + L_
        plsc.store_scatter(outbuf, [rvec, pos1], mv,
                           mask=((iota + L_) < cnt) & (pos1 < K_TOP_))
        return i + cnt, _vmax_splat(nxt, iota)

    lax.while_loop(emit_cond, emit, (jnp.int32(0), mv0))


def _sc_topk_body(x_hbm, out_hbm, rowbuf0, rowbuf1, candbuf, blkidx, outbuf,
                  sem0, sem1):
    wid = lax.axis_index("s") * N_CORES_ + lax.axis_index("c")
    iota = lax.iota(jnp.int32, L_)
    neg = jnp.full((L_,), -jnp.inf, jnp.float32)
    row0 = wid * RPW_

    bufs = [rowbuf0, rowbuf1]
    sems = [sem0, sem1]
    cps = [None] * RPW_
    cps[0] = pltpu.async_copy(x_hbm.at[row0], rowbuf0.at[pl.ds(0, N_)], sem0)
    for r in range(RPW_):
        cps[r].wait()
        if r + 1 < RPW_:
            cps[r + 1] = pltpu.async_copy(
                x_hbm.at[row0 + r + 1],
                bufs[(r + 1) % 2].at[pl.ds(0, N_)], sems[(r + 1) % 2])
        _process_row(bufs[r % 2], candbuf, blkidx, outbuf, r, iota, neg)
    pltpu.sync_copy(outbuf, out_hbm.at[pl.ds(row0, RPW_)])


@functools.lru_cache(maxsize=1)
def _build_sc_topk():
    return pl.kernel(
        _sc_topk_body,
        out_type=jax.ShapeDtypeStruct((ROWS_, K_TOP_), jnp.float32),
        mesh=plsc.VectorSubcoreMesh(core_axis_name="c", subcore_axis_name="s",
                                    num_cores=N_CORES_,
                                    num_subcores=N_SUBCORES_),
        scratch_types=[
            pltpu.VMEM((N_ + L_,), jnp.float32),
            pltpu.VMEM((N_ + L_,), jnp.float32),
            pltpu.VMEM((N_ + L_,), jnp.float32),
            pltpu.VMEM((NCELLS_ + L_,), jnp.int32),
            pltpu.VMEM((RPW_, K_TOP_), jnp.float32),
            pltpu.SemaphoreType.DMA,
            pltpu.SemaphoreType.DMA,
        ],
        compiler_params=pltpu.CompilerParams(needs_layout_passes=False),
    )


def kernel(inputs):
    return _build_sc_topk()(inputs)


# speedup vs baseline: 2.2910x; 1.0957x over previous
import functools
import jax
import jax.numpy as jnp
from jax import lax
from jax.experimental import pallas as pl
from jax.experimental.pallas import tpu as pltpu
from jax.experimental.pallas import tpu_sc as plsc

K_TOP_ = 32
L_ = 16
N_ = 32768
ROWS_ = 128
N_CORES_ = 2
N_SUBCORES_ = 16
N_WORKERS_ = N_CORES_ * N_SUBCORES_
RPW_ = ROWS_ // N_WORKERS_
NG_ = 32
NV_ = N_ // L_
NCELLS_ = NG_ * L_
CELL_ = N_ // NCELLS_

_GATHER_DNUMS_ = lax.GatherDimensionNumbers(
    offset_dims=(), collapsed_slice_dims=(0,), start_index_map=(0,))


def _lane_perm(v, idx):
    return lax.gather(v, idx[:, None], _GATHER_DNUMS_, slice_sizes=(1,),
                      mode=lax.GatherScatterMode.PROMISE_IN_BOUNDS,
                      unique_indices=True)


def _vmax_splat(v, iota):
    for s in (1, 2, 4, 8):
        v = jnp.maximum(v, _lane_perm(v, iota ^ s))
    return v


def _process_row(rowbuf, candbuf, blkidx, outbuf, rslot, iota, neg):
    rowbuf[pl.ds(N_, L_)] = neg

    half_words = (NV_ // 2) * L_

    def p1_pass(h):
        def p1(i, accs):
            base = h * half_words + i * (L_ * L_)
            return tuple(
                jnp.maximum(accs[g], rowbuf[pl.ds(base + g * L_, L_)])
                for g in range(L_))

        return lax.fori_loop(0, NV_ // 2 // L_, p1, (neg,) * L_)

    accs_a = p1_pass(0)
    accs_b = p1_pass(1)
    accs = list(accs_a) + list(accs_b)

    ahalf = accs_a[0]
    for g in range(1, L_):
        ahalf = jnp.maximum(ahalf, accs_a[g])
    bhalf = accs_b[0]
    for g in range(1, L_):
        bhalf = jnp.maximum(bhalf, accs_b[g])
    t0v = -_vmax_splat(-jnp.minimum(ahalf, bhalf), iota)

    thi = _vmax_splat(jnp.maximum(ahalf, bhalf), iota)
    k32 = jnp.int32(K_TOP_)
    for _ in range(8):
        tmid = 0.5 * (t0v + thi)
        cacc = jnp.zeros((L_,), jnp.int32)
        for g in range(NG_):
            cacc = cacc + (accs[g] >= tmid).astype(jnp.int32)
        ok = jnp.sum(cacc) >= k32
        t0v = jnp.where(ok, tmid, t0v)
        thi = jnp.where(ok, thi, tmid)

    cnts = [jnp.sum((accs[g] >= t0v).astype(jnp.int32)) for g in range(NG_)]
    nb = jnp.int32(0)
    for g in range(NG_):
        msk = accs[g] >= t0v
        plsc.store_compressed(blkidx.at[pl.ds(nb, L_)], g * L_ + iota,
                              mask=msk)
        nb = nb + cnts[g]
    blkidx[pl.ds(nb, L_)] = jnp.full((L_,), N_, jnp.int32)

    stride_w = NCELLS_ // 2
    U2B = 4
    nfull = (nb + (L_ - 1)) // L_

    def p2b(gi, ptr):
        idvec = blkidx[pl.ds(gi * L_, L_)]
        base0 = idvec + jnp.where(idvec >= stride_w,
                                  half_words - stride_w, 0)

        def inner(k4, carry):
            ptr, base = carry
            idxs = [jnp.minimum(base + q * stride_w, N_) for q in range(U2B)]
            vs = [plsc.load_gather(rowbuf, [ix]) for ix in idxs]
            ms = [v >= t0v for v in vs]
            cs = [jnp.sum(m.astype(jnp.int32)) for m in ms]
            for q in range(U2B):
                plsc.store_compressed(candbuf.at[pl.ds(ptr, L_)], vs[q],
                                      mask=ms[q])
                ptr = ptr + cs[q]
            return ptr, base + U2B * stride_w

        ptr, _ = lax.fori_loop(0, CELL_ // U2B, inner, (ptr, base0))
        return ptr

    ptr = lax.fori_loop(0, nfull, p2b, jnp.int32(0))
    SORTCAP = 8
    for j in range(SORTCAP):
        candbuf[pl.ds(ptr + j * L_, L_)] = neg
    ncv = ptr // L_ + 1

    rvec = jnp.full((L_,), rslot, jnp.int32)
    rev_idx = (L_ - 1) - iota

    def rev(v):
        return _lane_perm(v, rev_idx)

    def sdesc(v):
        return rev(jnp.sort(v))

    def merge16(a, b):
        rb = rev(b)
        return sdesc(jnp.maximum(a, rb)), sdesc(jnp.minimum(a, rb))

    def merge32(p, q):
        a0, a1 = p
        b0, b1 = q
        e0 = jnp.maximum(a0, rev(b1))
        e1 = jnp.maximum(a1, rev(b0))
        g0 = jnp.maximum(e0, e1)
        g1 = jnp.minimum(e0, e1)
        return sdesc(g0), sdesc(g1)

    @pl.when(ncv <= SORTCAP)
    def _():
        s = [sdesc(candbuf[pl.ds(j * L_, L_)]) for j in range(SORTCAP)]
        m01 = merge16(s[0], s[1])
        m23 = merge16(s[2], s[3])
        m45 = merge16(s[4], s[5])
        m67 = merge16(s[6], s[7])
        f0, f1 = merge32(merge32(m01, m23), merge32(m45, m67))
        plsc.store_scatter(outbuf, [rvec, iota], f0)
        plsc.store_scatter(outbuf, [rvec, iota + L_], f1)

    @pl.when(ncv > SORTCAP)
    def _():
        def pa(j, acc):
            return jnp.maximum(acc, candbuf[pl.ds(j * L_, L_)])

        mv0 = _vmax_splat(lax.fori_loop(0, ncv, pa, neg), iota)

        def emit_cond(carry):
            return carry[0] < K_TOP_

        def emit(carry):
            i, mv = carry

            def pb(j, c):
                cntv, nxt = c
                v = candbuf[pl.ds(j * L_, L_)]
                eq = v == mv
                vnew = jnp.where(eq, neg, v)
                candbuf[pl.ds(j * L_, L_)] = vnew
                return cntv + eq.astype(jnp.int32), jnp.maximum(nxt, vnew)

            cntv, nxt = lax.fori_loop(0, ncv, pb,
                                      (jnp.zeros((L_,), jnp.int32), neg))
            cnt = jnp.sum(cntv)
            pos0 = iota + i
            plsc.store_scatter(outbuf, [rvec, pos0], mv,
                               mask=(iota < cnt) & (pos0 < K_TOP_))
            pos1 = pos0 + L_
            plsc.store_scatter(outbuf, [rvec, pos1], mv,
                               mask=((iota + L_) < cnt) & (pos1 < K_TOP_))
            return i + cnt, _vmax_splat(nxt, iota)

        lax.while_loop(emit_cond, emit, (jnp.int32(0), mv0))


def _sc_topk_body(x_hbm, out_hbm, rowbuf0, rowbuf1, candbuf, blkidx, outbuf,
                  sem0, sem1):
    wid = lax.axis_index("s") * N_CORES_ + lax.axis_index("c")
    iota = lax.iota(jnp.int32, L_)
    neg = jnp.full((L_,), -jnp.inf, jnp.float32)
    row0 = wid * RPW_

    bufs = [rowbuf0, rowbuf1]
    sems = [sem0, sem1]
    cps = [None] * RPW_
    cps[0] = pltpu.async_copy(x_hbm.at[row0], rowbuf0.at[pl.ds(0, N_)], sem0)
    for r in range(RPW_):
        cps[r].wait()
        if r + 1 < RPW_:
            cps[r + 1] = pltpu.async_copy(
                x_hbm.at[row0 + r + 1],
                bufs[(r + 1) % 2].at[pl.ds(0, N_)], sems[(r + 1) % 2])
        _process_row(bufs[r % 2], candbuf, blkidx, outbuf, r, iota, neg)
    pltpu.sync_copy(outbuf, out_hbm.at[pl.ds(row0, RPW_)])


@functools.lru_cache(maxsize=1)
def _build_sc_topk():
    return pl.kernel(
        _sc_topk_body,
        out_type=jax.ShapeDtypeStruct((ROWS_, K_TOP_), jnp.float32),
        mesh=plsc.VectorSubcoreMesh(core_axis_name="c", subcore_axis_name="s",
                                    num_cores=N_CORES_,
                                    num_subcores=N_SUBCORES_),
        scratch_types=[
            pltpu.VMEM((N_ + L_,), jnp.float32),
            pltpu.VMEM((N_ + L_,), jnp.float32),
            pltpu.VMEM((N_ + 8 * L_,), jnp.float32),
            pltpu.VMEM((NCELLS_ + L_,), jnp.int32),
            pltpu.VMEM((RPW_, K_TOP_), jnp.float32),
            pltpu.SemaphoreType.DMA,
            pltpu.SemaphoreType.DMA,
        ],
        compiler_params=pltpu.CompilerParams(needs_layout_passes=False),
    )


def kernel(inputs):
    return _build_sc_topk()(inputs)
